# baseline (speedup 1.0000x reference)
import functools

import jax
import jax.numpy as jnp
from jax.experimental import pallas as pl
from jax.experimental.pallas import tpu as pltpu

N_ITERS = 3
SUB = 8
NPREF = 4


def _sigmoid(x):
    return 1.0 / (1.0 + jnp.exp(-x))


def _scores(f2, qb):
    return jax.lax.dot_general(qb, f2, (((1,), (1,)), ((), ())),
                               preferred_element_type=jnp.float32)


def _softmax(E, GT):
    m = jnp.max(jnp.where(GT, E, -jnp.inf), axis=1, keepdims=True)
    ex = jnp.where(GT, jnp.exp(E - m), 0.0)
    den = jnp.sum(ex, axis=1, keepdims=True)
    return ex, den


def _readout(ex, den, f2):
    rsum = jnp.dot(ex, f2, preferred_element_type=jnp.float32)
    return rsum * (1.0 / den)


def _body(feat_hbm, fg_ref, wqT_ref, wrT_ref, whhT_ref, b_ref, out_ref,
          h_ref, c_ref, q_ref, r_ref, fv_ref, cp_sem, *, S, D, B):
    i = pl.program_id(0)
    CH = SUB * S
    nch = B // SUB

    def _chunk_copy(c):
        return pltpu.make_async_copy(
            feat_hbm.at[pl.ds(c * CH, CH), :],
            fv_ref.at[pl.ds(c * CH, CH), :],
            cp_sem.at[c % NPREF],
        )

    @pl.when(i == 0)
    def _prime():
        h_ref[...] = jnp.zeros_like(h_ref)
        c_ref[...] = jnp.zeros_like(c_ref)
        q_ref[...] = jnp.zeros_like(q_ref)
        r_ref[...] = jnp.zeros_like(r_ref)
        for c in range(NPREF):
            _chunk_copy(c).start()

    h = h_ref[...]
    c = c_ref[...]
    gates = (
        jnp.dot(q_ref[...], wqT_ref[...], preferred_element_type=jnp.float32)
        + jnp.dot(r_ref[...], wrT_ref[...], preferred_element_type=jnp.float32)
        + jnp.dot(h, whhT_ref[...], preferred_element_type=jnp.float32)
        + b_ref[...]
    )
    ig = _sigmoid(gates[:, 0 * D:1 * D])
    fg = _sigmoid(gates[:, 1 * D:2 * D])
    gg = jnp.tanh(gates[:, 2 * D:3 * D])
    og = _sigmoid(gates[:, 3 * D:4 * D])
    c_new = fg * c + ig * gg
    h_new = og * jnp.tanh(c_new)
    h_ref[...] = h_new
    c_ref[...] = c_new
    q_ref[...] = h_new
    qb_all = h_new

    row_seg = jax.lax.broadcasted_iota(jnp.int32, (SUB, CH), 1) // S
    row_k = jax.lax.broadcasted_iota(jnp.int32, (SUB, CH), 0)
    GT = row_seg == row_k

    Es, sms, readouts = {}, {}, [None] * nch
    for c in range(nch + 2):
        if c < nch:
            @pl.when(i == 0)
            def _dma_step(c=c):
                _chunk_copy(c).wait()
                if c + NPREF < nch:
                    _chunk_copy(c + NPREF).start()
            f2 = fv_ref[c * CH:(c + 1) * CH, :]
            Es[c] = _scores(f2, qb_all[c * SUB:(c + 1) * SUB, :])
        if 0 <= c - 1 < nch:
            sms[c - 1] = _softmax(Es.pop(c - 1), GT)
        if 0 <= c - 2 < nch:
            ex, den = sms.pop(c - 2)
            readouts[c - 2] = _readout(
                ex, den, fv_ref[(c - 2) * CH:(c - 1) * CH, :])
    readout = jnp.concatenate(readouts, axis=0)

    r_ref[...] = readout
    out_ref[...] = jnp.concatenate([qb_all, readout, fg_ref[...]], axis=1)


def kernel(feat_atom, sizes_atom, feat_global, W_ih, W_hh, b_ih, b_hh):
    del sizes_atom
    N, D = feat_atom.shape
    B = feat_global.shape[0]
    S = N // B

    wT = W_ih.T
    wqT = wT[:D, :]
    wrT = wT[D:, :]
    whhT = W_hh.T
    b = (b_ih + b_hh).reshape(1, 4 * D)

    out = pl.pallas_call(
        functools.partial(_body, S=S, D=D, B=B),
        grid=(N_ITERS,),
        in_specs=[
            pl.BlockSpec(memory_space=pltpu.MemorySpace.HBM),
            pl.BlockSpec((B, D), lambda i: (0, 0)),
            pl.BlockSpec((D, 4 * D), lambda i: (0, 0)),
            pl.BlockSpec((D, 4 * D), lambda i: (0, 0)),
            pl.BlockSpec((D, 4 * D), lambda i: (0, 0)),
            pl.BlockSpec((1, 4 * D), lambda i: (0, 0)),
        ],
        out_specs=pl.BlockSpec((B, 3 * D), lambda i: (0, 0)),
        out_shape=jax.ShapeDtypeStruct((B, 3 * D), jnp.float32),
        scratch_shapes=[
            pltpu.VMEM((B, D), jnp.float32),
            pltpu.VMEM((B, D), jnp.float32),
            pltpu.VMEM((B, D), jnp.float32),
            pltpu.VMEM((B, D), jnp.float32),
            pltpu.VMEM((N, D), jnp.float32),
            pltpu.SemaphoreType.DMA((NPREF,)),
        ],
        compiler_params=pltpu.CompilerParams(
            dimension_semantics=("arbitrary",),
        ),
    )(feat_atom, feat_global, wqT, wrT, whhT, b)
    return out


# speedup vs baseline: 1.0733x; 1.0733x over previous
import functools

import jax
import jax.numpy as jnp
from jax.experimental import pallas as pl
from jax.experimental.pallas import tpu as pltpu

N_ITERS = 3
BBLK = 40
SUB = 8
NSUB = BBLK // SUB


def _sigmoid(x):
    return 1.0 / (1.0 + jnp.exp(-x))


def _scores(f2, qb):
    return jax.lax.dot_general(qb, f2, (((1,), (1,)), ((), ())),
                               preferred_element_type=jnp.float32)


def _softmax(E, GT):
    m = jnp.max(jnp.where(GT, E, -jnp.inf), axis=1, keepdims=True)
    ex = jnp.where(GT, jnp.exp(E - m), 0.0)
    den = jnp.sum(ex, axis=1, keepdims=True)
    return ex, den


def _readout(ex, den, f2):
    rsum = jnp.dot(ex, f2, preferred_element_type=jnp.float32)
    return rsum * (1.0 / den)


def _body(feat_hbm, fg_ref, wcat_ref, b_ref, out_ref,
          h_ref, c_ref, q_ref, r_ref, fv_ref, cp_sem, *, S, D):
    i = pl.program_id(0)
    j = pl.program_id(1)
    nblk = pl.num_programs(1)
    RB = BBLK * S

    def _chunk_copy(jj):
        return pltpu.make_async_copy(
            feat_hbm.at[pl.ds(jj * RB, RB), :],
            fv_ref.at[pl.ds(jj * RB, RB), :],
            cp_sem.at[jj % 2],
        )

    @pl.when(jnp.logical_and(i == 0, j == 0))
    def _prime():
        _chunk_copy(0).start()
        _chunk_copy(1).start()

    @pl.when(jnp.logical_and(i == 0, jnp.logical_and(j >= 1, j < nblk - 1)))
    def _prefetch():
        _chunk_copy(j + 1).start()

    @pl.when(i == 0)
    def _wait():
        _chunk_copy(j).wait()

    @pl.when(jnp.logical_and(i == 0, j == 0))
    def _init():
        h_ref[...] = jnp.zeros_like(h_ref)
        c_ref[...] = jnp.zeros_like(c_ref)
        q_ref[...] = jnp.zeros_like(q_ref)
        r_ref[...] = jnp.zeros_like(r_ref)

    @pl.when(j == 0)
    def _lstm():
        h = h_ref[...]
        c = c_ref[...]
        x = jnp.concatenate([q_ref[...], r_ref[...], h], axis=1)
        gates = jnp.dot(x, wcat_ref[...],
                        preferred_element_type=jnp.float32) + b_ref[...]
        ig = _sigmoid(gates[:, 0 * D:1 * D])
        fg = _sigmoid(gates[:, 1 * D:2 * D])
        gg = jnp.tanh(gates[:, 2 * D:3 * D])
        og = _sigmoid(gates[:, 3 * D:4 * D])
        c_new = fg * c + ig * gg
        h_new = og * jnp.tanh(c_new)
        h_ref[...] = h_new
        c_ref[...] = c_new
        q_ref[...] = h_new

    qb_all = q_ref[pl.ds(j * BBLK, BBLK), :]
    Rsub = SUB * S
    row_seg = jax.lax.broadcasted_iota(jnp.int32, (SUB, Rsub), 1) // S
    row_k = jax.lax.broadcasted_iota(jnp.int32, (SUB, Rsub), 0)
    GT = row_seg == row_k
    f2s = [fv_ref[pl.ds(j * RB + h * SUB * S, SUB * S), :] for h in range(NSUB)]
    Es = [_scores(f2s[h], qb_all[h * SUB:(h + 1) * SUB, :]) for h in range(NSUB)]
    sms = [_softmax(Es[h], GT) for h in range(NSUB)]
    readouts = [_readout(ex, den, f2s[h]) for h, (ex, den) in enumerate(sms)]
    readout = jnp.concatenate(readouts, axis=0)

    r_ref[pl.ds(j * BBLK, BBLK), :] = readout
    out_ref[...] = jnp.concatenate([qb_all, readout, fg_ref[...]], axis=1)


def kernel(feat_atom, sizes_atom, feat_global, W_ih, W_hh, b_ih, b_hh):
    del sizes_atom
    N, D = feat_atom.shape
    B = feat_global.shape[0]
    S = N // B
    R = BBLK * S
    nblk = B // BBLK

    wcat = jnp.concatenate([W_ih.T, W_hh.T], axis=0)
    b = (b_ih + b_hh).reshape(1, 4 * D)

    grid = (N_ITERS, nblk)
    out = pl.pallas_call(
        functools.partial(_body, S=S, D=D),
        grid=grid,
        in_specs=[
            pl.BlockSpec(memory_space=pltpu.MemorySpace.HBM),
            pl.BlockSpec((BBLK, D), lambda i, j: (j, 0)),
            pl.BlockSpec((3 * D, 4 * D), lambda i, j: (0, 0)),
            pl.BlockSpec((1, 4 * D), lambda i, j: (0, 0)),
        ],
        out_specs=pl.BlockSpec((BBLK, 3 * D), lambda i, j: (j, 0)),
        out_shape=jax.ShapeDtypeStruct((B, 3 * D), jnp.float32),
        scratch_shapes=[
            pltpu.VMEM((B, D), jnp.float32),
            pltpu.VMEM((B, D), jnp.float32),
            pltpu.VMEM((B, D), jnp.float32),
            pltpu.VMEM((B, D), jnp.float32),
            pltpu.VMEM((N, D), jnp.float32),
            pltpu.SemaphoreType.DMA((2,)),
        ],
        compiler_params=pltpu.CompilerParams(
            dimension_semantics=("arbitrary", "arbitrary"),
        ),
    )(feat_atom, feat_global, wcat, b)
    return out
